# baseline (speedup 1.0000x reference)
import jax
import jax.numpy as jnp
from jax import lax
from jax.experimental import pallas as pl
from jax.experimental.pallas import tpu as pltpu
from jax.experimental.pallas import tpu_sc as plsc

N = 100000
K = 10000
F = 256

R = 80
HS = (32, 48)
C = N // R
NC = 2
NS = 16
NW = NC * NS
CPW = (C + NW - 1) // NW
CREM = C % NW
WINC = CPW + 1
NB = 6


def _splat(s16, rr):
    return lax.gather(
        s16,
        jnp.full((16, 1), rr, jnp.int32),
        dimension_numbers=lax.GatherDimensionNumbers(
            offset_dims=(),
            collapsed_slice_dims=(0,),
            start_index_map=(0,),
        ),
        slice_sizes=(1,),
        mode=lax.GatherScatterMode.PROMISE_IN_BOUNDS,
    )


def _scale_half(rows_v, b, s_all_v, s_base, half):
    g0 = sum(HS[:half]) // 16
    for g in range(g0, g0 + HS[half] // 16):
        s16 = s_all_v[pl.ds(s_base + g * 16, 16)]
        for rr in range(16):
            sv = _splat(s16, rr)
            r = g * 16 + rr
            for cpart in range(F // 16):
                sl = pl.ds(cpart * 16, 16)
                rows_v[b, r, sl] = rows_v[b, r, sl] * sv


def _lift_body(x_hbm, idx_hbm, s_hbm, out_hbm,
               idx_all_v, s_all_v, rows_v, sems_in, sems_out):
    wid = lax.axis_index("s") * NC + lax.axis_index("c")
    nk = jnp.where(wid < CREM, CPW, CPW - 1)
    ch0 = (CPW - 1) * wid + jnp.minimum(wid, CREM)
    win = jnp.minimum(ch0, C - WINC)
    k0 = ch0 - win

    def fire_gather(kk):
        b = kk % NB
        for j in range(2):
            pltpu.async_copy(
                x_hbm.at[idx_all_v.at[pl.ds((k0 + kk) * R + j * (R // 2), R // 2)]],
                rows_v.at[b, pl.ds(j * (R // 2), R // 2)],
                sems_in.at[b],
            )

    def drain_in(kk):
        b = kk % NB
        for _ in range(2):
            pltpu.make_async_copy(
                x_hbm.at[pl.ds(0, R // 2)],
                rows_v.at[b, pl.ds(0, R // 2)],
                sems_in.at[b],
            ).wait()

    def fire_out(kk):
        b = kk % NB
        ch = ch0 + kk
        pltpu.async_copy(
            rows_v.at[b], out_hbm.at[pl.ds(ch * R, R)], sems_out.at[b]
        )

    def drain_out(kk):
        b = kk % NB
        pltpu.make_async_copy(
            rows_v.at[b], out_hbm.at[pl.ds(0, R)], sems_out.at[b]
        ).wait()

    pltpu.sync_copy(idx_hbm.at[pl.ds(win * R, WINC * R)], idx_all_v)
    pltpu.sync_copy(s_hbm.at[pl.ds(win * R, WINC * R)], s_all_v)

    for kk in range(NB - 2):
        fire_gather(kk)

    def loop_body(k, carry):
        kk = k + (NB - 2)

        @pl.when(kk < nk)
        def _():
            @pl.when(k >= 2)
            def _():
                drain_out(kk - NB)

            fire_gather(kk)

        drain_in(k)
        b = k % NB
        _scale_half(rows_v, b, s_all_v, (k0 + k) * R, 0)
        _scale_half(rows_v, b, s_all_v, (k0 + k) * R, 1)
        fire_out(k)
        return carry

    lax.fori_loop(0, nk, loop_body, 0)

    def tail_body(t, carry):
        @pl.when(t >= nk - NB)
        def _():
            drain_out(t)

        return carry

    lax.fori_loop(0, nk, tail_body, 0)


@jax.jit
def kernel(x_pool, cluster_index, s_values):
    idx = cluster_index.astype(jnp.int32)
    s = s_values

    mesh = plsc.VectorSubcoreMesh(core_axis_name="c", subcore_axis_name="s")
    lift = pl.kernel(
        _lift_body,
        out_type=jax.ShapeDtypeStruct((N, F), jnp.float32),
        mesh=mesh,
        scratch_types=[
            pltpu.VMEM((WINC * R,), jnp.int32),
            pltpu.VMEM((WINC * R,), jnp.float32),
            pltpu.VMEM((NB, R, F), jnp.float32),
            pltpu.SemaphoreType.DMA((NB,)),
            pltpu.SemaphoreType.DMA((NB,)),
        ],
    )
    return lift(x_pool, idx, s)


# speedup vs baseline: 1.0811x; 1.0811x over previous
import jax
import jax.numpy as jnp
from jax import lax
from jax.experimental import pallas as pl
from jax.experimental.pallas import tpu as pltpu
from jax.experimental.pallas import tpu_sc as plsc

N = 100000
K = 10000
F = 256

R = 80
HS = (32, 48)
C = N // R
NC = 2
NS = 16
NW = NC * NS
CPW = (C + NW - 1) // NW
CREM = C % NW
WINC = CPW + 1
NB = 6


def _splat(s16, rr):
    return lax.gather(
        s16,
        jnp.full((16, 1), rr, jnp.int32),
        dimension_numbers=lax.GatherDimensionNumbers(
            offset_dims=(),
            collapsed_slice_dims=(0,),
            start_index_map=(0,),
        ),
        slice_sizes=(1,),
        mode=lax.GatherScatterMode.PROMISE_IN_BOUNDS,
    )


def _scale_half(rows_v, b, s_all_v, s_base, half):
    g0 = sum(HS[:half]) // 16
    for g in range(g0, g0 + HS[half] // 16):
        s16 = s_all_v[pl.ds(s_base + g * 16, 16)]
        for rr in range(16):
            sv = _splat(s16, rr)
            r = g * 16 + rr
            for cpart in range(F // 16):
                sl = pl.ds(cpart * 16, 16)
                rows_v[b, r, sl] = rows_v[b, r, sl] * sv


def _lift_body(x_hbm, idx_hbm, s_hbm, out_hbm,
               idx_all_v, s_all_v, rows_v, sems_in, sems_out):
    wid = lax.axis_index("s") * NC + lax.axis_index("c")
    nk = jnp.where(wid < CREM, CPW, CPW - 1)
    ch0 = (CPW - 1) * wid + jnp.minimum(wid, CREM)
    win = jnp.minimum(ch0, C - WINC)
    k0 = ch0 - win

    def fire_gather(kk):
        b = kk % NB
        pltpu.async_copy(
            x_hbm.at[idx_all_v.at[pl.ds((k0 + kk) * R, R)]],
            rows_v.at[b],
            sems_in.at[b],
        )

    def drain_in(kk):
        b = kk % NB
        pltpu.make_async_copy(
            x_hbm.at[pl.ds(0, R)], rows_v.at[b], sems_in.at[b]
        ).wait()

    def fire_out(kk):
        b = kk % NB
        ch = ch0 + kk
        pltpu.async_copy(
            rows_v.at[b], out_hbm.at[pl.ds(ch * R, R)], sems_out.at[b]
        )

    def drain_out(kk):
        b = kk % NB
        pltpu.make_async_copy(
            rows_v.at[b], out_hbm.at[pl.ds(0, R)], sems_out.at[b]
        ).wait()

    pltpu.sync_copy(idx_hbm.at[pl.ds(win * R, WINC * R)], idx_all_v)
    pltpu.sync_copy(s_hbm.at[pl.ds(win * R, WINC * R)], s_all_v)

    for kk in range(NB - 2):
        fire_gather(kk)

    def loop_body(k, carry):
        kk = k + (NB - 2)

        @pl.when(kk < nk)
        def _():
            @pl.when(k >= 2)
            def _():
                drain_out(kk - NB)

            fire_gather(kk)

        drain_in(k)
        b = k % NB
        _scale_half(rows_v, b, s_all_v, (k0 + k) * R, 0)
        _scale_half(rows_v, b, s_all_v, (k0 + k) * R, 1)
        fire_out(k)
        return carry

    lax.fori_loop(0, nk, loop_body, 0)

    def tail_body(t, carry):
        @pl.when(t >= nk - NB)
        def _():
            drain_out(t)

        return carry

    lax.fori_loop(0, nk, tail_body, 0)


@jax.jit
def kernel(x_pool, cluster_index, s_values):
    idx = cluster_index.astype(jnp.int32)
    s = s_values

    mesh = plsc.VectorSubcoreMesh(core_axis_name="c", subcore_axis_name="s")
    lift = pl.kernel(
        _lift_body,
        out_type=jax.ShapeDtypeStruct((N, F), jnp.float32),
        mesh=mesh,
        scratch_types=[
            pltpu.VMEM((WINC * R,), jnp.int32),
            pltpu.VMEM((WINC * R,), jnp.float32),
            pltpu.VMEM((NB, R, F), jnp.float32),
            pltpu.SemaphoreType.DMA((NB,)),
            pltpu.SemaphoreType.DMA((NB,)),
        ],
    )
    return lift(x_pool, idx, s)
